# baseline (speedup 1.0000x reference)
import jax
import jax.numpy as jnp
from jax.experimental import pallas as pl
from jax.experimental.pallas import tpu as pltpu

_TN = 1024


def _body(x_ref, w_ref, b_ref, o_ref):
    k = pl.program_id(1)
    contrib = jax.lax.dot_general(
        x_ref[0], w_ref[k],
        (((1,), (1,)), ((), ())),
        preferred_element_type=jnp.float32)
    contrib = contrib + b_ref[k][None, :]

    @pl.when(k == 0)
    def _init():
        o_ref[...] = contrib

    @pl.when(k != 0)
    def _accum():
        o_ref[...] = o_ref[...] + contrib


def kernel(inputs, weights, bias):
    k_tot, n, in_ch = inputs.shape
    out_ch = weights.shape[1]
    tn = min(_TN, n)
    return pl.pallas_call(
        _body,
        grid=(n // tn, k_tot),
        in_specs=[
            pl.BlockSpec((1, tn, in_ch), lambda i, k: (k, i, 0)),
            pl.BlockSpec((k_tot, out_ch, in_ch), lambda i, k: (0, 0, 0)),
            pl.BlockSpec((k_tot, out_ch), lambda i, k: (0, 0)),
        ],
        out_specs=pl.BlockSpec((tn, out_ch), lambda i, k: (i, 0)),
        out_shape=jax.ShapeDtypeStruct((n, out_ch), jnp.float32),
        compiler_params=pltpu.CompilerParams(
            dimension_semantics=("parallel", "arbitrary"),
        ),
    )(inputs, weights, bias)


# speedup vs baseline: 2.5365x; 2.5365x over previous
import jax
import jax.numpy as jnp
from jax.experimental import pallas as pl
from jax.experimental.pallas import tpu as pltpu

_TN = 1024


def _body(x_ref, w_ref, b_ref, o_ref):
    k_tot = w_ref.shape[0]
    tn, out_ch = o_ref.shape
    acc = jnp.zeros((tn, out_ch), jnp.float32)
    for k in range(k_tot):
        acc = acc + jax.lax.dot_general(
            x_ref[k], w_ref[k],
            (((1,), (1,)), ((), ())),
            preferred_element_type=jnp.float32)
    o_ref[...] = acc + jnp.sum(b_ref[...], axis=0)[None, :]


def kernel(inputs, weights, bias):
    k_tot, n, in_ch = inputs.shape
    out_ch = weights.shape[1]
    tn = min(_TN, n)
    return pl.pallas_call(
        _body,
        grid=(n // tn,),
        in_specs=[
            pl.BlockSpec((k_tot, tn, in_ch), lambda i: (0, i, 0)),
            pl.BlockSpec((k_tot, out_ch, in_ch), lambda i: (0, 0, 0)),
            pl.BlockSpec((k_tot, out_ch), lambda i: (0, 0)),
        ],
        out_specs=pl.BlockSpec((tn, out_ch), lambda i: (i, 0)),
        out_shape=jax.ShapeDtypeStruct((n, out_ch), jnp.float32),
        compiler_params=pltpu.CompilerParams(
            dimension_semantics=("parallel",),
        ),
    )(inputs, weights, bias)
